# baseline (speedup 1.0000x reference)
import functools

import jax
import jax.numpy as jnp
from jax import lax
from jax.experimental import pallas as pl
from jax.experimental.pallas import tpu as pltpu
from jax.experimental.pallas import tpu_sc as plsc

N_NODES = 10000
D_FEAT = 128
D_HID = 64
NC, NS = 2, 16
NW = NC * NS
N_PAD = 10240
RPT = N_PAD // NS
CHUNK = 128
NB = 8

MB = 512
GRID_M = N_PAD // MB


def _sc_degree(dst2):
    total_chunks = dst2.shape[0]
    k_ch = total_chunks // NW
    n_grp = k_ch // NB
    mesh = plsc.VectorSubcoreMesh(core_axis_name="c", subcore_axis_name="s")

    @functools.partial(
        pl.kernel,
        out_type=jax.ShapeDtypeStruct((NC, N_PAD), jnp.float32),
        mesh=mesh,
        scratch_types=[
            pltpu.VMEM((k_ch, CHUNK), jnp.int32),
            pltpu.VMEM((CHUNK,), jnp.float32),
            pltpu.VMEM_SHARED((N_PAD,), jnp.float32),
            pltpu.VMEM((RPT,), jnp.float32),
        ] + [pltpu.SemaphoreType.DMA] * NB,
    )
    def deg_kernel(dst_hbm, out_hbm, idx_all, ones_v, deg_sh, zero_v, *sems):
        c = lax.axis_index("c")
        s = lax.axis_index("s")
        ch0 = (c * NS + s) * k_ch
        r0 = s * RPT
        for i in range(CHUNK // 16):
            ones_v[pl.ds(i * 16, 16)] = jnp.ones((16,), jnp.float32)
        for i in range(RPT // 16):
            zero_v[pl.ds(i * 16, 16)] = jnp.zeros((16,), jnp.float32)
        pltpu.sync_copy(dst_hbm.at[pl.ds(ch0, k_ch)], idx_all)
        pltpu.sync_copy(zero_v, deg_sh.at[pl.ds(r0, RPT)])
        plsc.subcore_barrier()

        def scat(k, b):
            pltpu.async_copy(ones_v, deg_sh.at[idx_all.at[k]], sems[b],
                             add=True)

        def grp(g, carry):
            for b in range(NB):
                @pl.when(g > 0)
                def _():
                    pltpu.make_async_copy(
                        ones_v, deg_sh.at[idx_all.at[0]], sems[b]).wait()
                scat(g * NB + b, b)
            return carry

        lax.fori_loop(0, n_grp, grp, 0)
        for b in range(NB):
            pltpu.make_async_copy(
                ones_v, deg_sh.at[idx_all.at[0]], sems[b]).wait()
        plsc.subcore_barrier()
        pltpu.sync_copy(deg_sh.at[pl.ds(r0, RPT)],
                        out_hbm.at[c, pl.ds(r0, RPT)])

    return deg_kernel(dst2)


NBP = 4


def _sc_propagate(hs_pad, src2, dst2, zeros2):
    total_chunks = src2.shape[0]
    k_ch = total_chunks // NW
    n_grp = k_ch // NBP
    mesh = plsc.VectorSubcoreMesh(core_axis_name="c", subcore_axis_name="s")

    @functools.partial(
        pl.kernel,
        out_type=jax.ShapeDtypeStruct((NC, N_PAD, D_HID), jnp.float32),
        mesh=mesh,
        scratch_types=[
            pltpu.VMEM((NBP, CHUNK), jnp.int32),
            pltpu.VMEM((NBP, CHUNK), jnp.int32),
            pltpu.VMEM((NBP, CHUNK, D_HID), jnp.float32),
            pltpu.VMEM_SHARED((N_PAD, D_HID), jnp.float32),
            pltpu.VMEM_SHARED((N_PAD, D_HID), jnp.float32),
        ] + [pltpu.SemaphoreType.DMA] * (4 * NBP),
        compiler_params=pltpu.CompilerParams(use_tc_tiling_on_sc=False),
    )
    def prop_kernel(hs_hbm, src_hbm, dst_hbm, zeros_hbm, out_hbm,
                    sidx, didx, rows, acc_sh, hs_sh, *sems):
        gsem = sems[:NBP]
        ssem = sems[NBP:2 * NBP]
        sisem = sems[2 * NBP:3 * NBP]
        disem = sems[3 * NBP:]
        c = lax.axis_index("c")
        s = lax.axis_index("s")
        ch0 = (c * NS + s) * k_ch
        r0 = s * RPT
        pltpu.sync_copy(zeros_hbm.at[pl.ds(r0, RPT)], acc_sh.at[pl.ds(r0, RPT)])
        pltpu.sync_copy(hs_hbm.at[pl.ds(r0, RPT)], hs_sh.at[pl.ds(r0, RPT)])
        plsc.subcore_barrier()

        def start_idx(k, b):
            pltpu.async_copy(src_hbm.at[ch0 + k], sidx.at[b], sisem[b])
            pltpu.async_copy(dst_hbm.at[ch0 + k], didx.at[b], disem[b])

        def wait_idx(b):
            pltpu.make_async_copy(src_hbm.at[0], sidx.at[b], sisem[b]).wait()
            pltpu.make_async_copy(dst_hbm.at[0], didx.at[b], disem[b]).wait()

        def start_gather(b):
            pltpu.async_copy(hs_sh.at[sidx.at[b]], rows.at[b], gsem[b])

        def wait_gather(b):
            pltpu.make_async_copy(
                hs_sh.at[sidx.at[0]], rows.at[b], gsem[b]).wait()

        def start_scatter(b):
            pltpu.async_copy(rows.at[b], acc_sh.at[didx.at[b]], ssem[b],
                             add=True)

        def wait_scatter(b):
            pltpu.make_async_copy(
                rows.at[b], acc_sh.at[didx.at[0]], ssem[b]).wait()

        for b in range(NBP):
            start_idx(b, b)
        for b in range(NBP):
            wait_idx(b)
            start_gather(b)

        def grp(g, carry):
            for b in range(NBP):
                wait_gather(b)
                start_scatter(b)
            for b in range(NBP):
                wait_scatter(b)

                @pl.when(g + 1 < n_grp)
                def _():
                    start_idx((g + 1) * NBP + b, b)
            for b in range(NBP):
                @pl.when(g + 1 < n_grp)
                def _():
                    wait_idx(b)
                    start_gather(b)
            return carry

        lax.fori_loop(0, n_grp, grp, 0)
        plsc.subcore_barrier()
        pltpu.sync_copy(acc_sh.at[pl.ds(r0, RPT)],
                        out_hbm.at[c, pl.ds(r0, RPT)])

    return prop_kernel(hs_pad, src2, dst2, zeros2)


def _tc_first(deg_parts_t, x_pad, W1):
    def body(deg_ref, x_ref, w_ref, hs_ref, dinv_ref):
        deg = deg_ref[...]
        degt = deg[:, 0:1] + deg[:, 1:2] + 1.0
        dinv = lax.rsqrt(jnp.maximum(degt, 1e-12))
        h = jnp.dot(x_ref[...], w_ref[...], preferred_element_type=jnp.float32)
        hs_ref[...] = dinv * h
        dinv_ref[...] = dinv

    return pl.pallas_call(
        body,
        out_shape=[
            jax.ShapeDtypeStruct((N_PAD, D_HID), jnp.float32),
            jax.ShapeDtypeStruct((N_PAD, 1), jnp.float32),
        ],
    )(deg_parts_t, x_pad, W1)


def _tc_mid(acc_parts, hs1, dinv, b1, W2):
    def body(acc_ref, hs_ref, dinv_ref, b_ref, w_ref, out_ref):
        agg = acc_ref[0] + acc_ref[1] + hs_ref[...]
        pre = dinv_ref[...] * agg + b_ref[...]
        h = jnp.maximum(pre, 0.0)
        h2 = jnp.dot(h, w_ref[...], preferred_element_type=jnp.float32)
        out_ref[...] = dinv_ref[...] * h2

    return pl.pallas_call(
        body,
        out_shape=jax.ShapeDtypeStruct((N_PAD, D_HID), jnp.float32),
    )(acc_parts, hs1, dinv, b1, W2)


def _tc_final(acc_parts, hs2, dinv, b2):
    def body(acc_ref, hs_ref, dinv_ref, b_ref, out_ref):
        agg = acc_ref[0] + acc_ref[1] + hs_ref[...]
        out_ref[...] = dinv_ref[...] * agg + b_ref[...]

    return pl.pallas_call(
        body,
        out_shape=jax.ShapeDtypeStruct((N_PAD, D_HID), jnp.float32),
    )(acc_parts, hs2, dinv, b2)


def kernel(features, edge_index, W1, b1, W2, b2):
    src = edge_index[0].astype(jnp.int32)
    dst = edge_index[1].astype(jnp.int32)
    e = src.shape[0]
    quantum = NW * CHUNK
    k_ch = -(-e // quantum)
    k_ch = -(-k_ch // NB) * NB
    e_pad = k_ch * quantum
    fill = (N_NODES + jnp.arange(e_pad - e, dtype=jnp.int32)
            % (N_PAD - N_NODES))
    src2 = jnp.concatenate([src, fill]).reshape(NW * k_ch, CHUNK)
    dst2 = jnp.concatenate([dst, fill]).reshape(NW * k_ch, CHUNK)

    x_pad = jnp.pad(features, ((0, N_PAD - N_NODES), (0, 0)))
    zeros2 = jnp.zeros((N_PAD, D_HID), jnp.float32)
    b1r = b1.reshape(1, D_HID)
    b2r = b2.reshape(1, D_HID)

    deg_parts = _sc_degree(dst2)
    hs1, dinv = _tc_first(deg_parts.T, x_pad, W1)
    acc1 = _sc_propagate(hs1, src2, dst2, zeros2)
    hs2 = _tc_mid(acc1, hs1, dinv, b1r, W2)
    acc2 = _sc_propagate(hs2, src2, dst2, zeros2)
    out = _tc_final(acc2, hs2, dinv, b2r)
    return out[:N_NODES]


# speedup vs baseline: 1.3957x; 1.3957x over previous
import functools

import jax
import jax.numpy as jnp
from jax import lax
from jax.experimental import pallas as pl
from jax.experimental.pallas import tpu as pltpu
from jax.experimental.pallas import tpu_sc as plsc

N_NODES = 10000
D_FEAT = 128
D_HID = 64
NC, NS = 2, 16
NW = NC * NS
N_PAD = 10240
RPT = N_PAD // NS
CHUNK = 128
NB = 8

MB = 2560
GRID_M = N_PAD // MB


def _sc_degree(dst2):
    total_chunks = dst2.shape[0]
    k_ch = total_chunks // NW
    n_grp = k_ch // NB
    mesh = plsc.VectorSubcoreMesh(core_axis_name="c", subcore_axis_name="s")

    @functools.partial(
        pl.kernel,
        out_type=jax.ShapeDtypeStruct((NC, N_PAD), jnp.float32),
        mesh=mesh,
        scratch_types=[
            pltpu.VMEM((k_ch, CHUNK), jnp.int32),
            pltpu.VMEM((CHUNK,), jnp.float32),
            pltpu.VMEM_SHARED((N_PAD,), jnp.float32),
            pltpu.VMEM((RPT,), jnp.float32),
        ] + [pltpu.SemaphoreType.DMA] * NB,
    )
    def deg_kernel(dst_hbm, out_hbm, idx_all, ones_v, deg_sh, zero_v, *sems):
        c = lax.axis_index("c")
        s = lax.axis_index("s")
        ch0 = (c * NS + s) * k_ch
        r0 = s * RPT
        for i in range(CHUNK // 16):
            ones_v[pl.ds(i * 16, 16)] = jnp.ones((16,), jnp.float32)
        for i in range(RPT // 16):
            zero_v[pl.ds(i * 16, 16)] = jnp.zeros((16,), jnp.float32)
        pltpu.sync_copy(dst_hbm.at[pl.ds(ch0, k_ch)], idx_all)
        pltpu.sync_copy(zero_v, deg_sh.at[pl.ds(r0, RPT)])
        plsc.subcore_barrier()

        def scat(k, b):
            pltpu.async_copy(ones_v, deg_sh.at[idx_all.at[k]], sems[b],
                             add=True)

        def grp(g, carry):
            for b in range(NB):
                @pl.when(g > 0)
                def _():
                    pltpu.make_async_copy(
                        ones_v, deg_sh.at[idx_all.at[0]], sems[b]).wait()
                scat(g * NB + b, b)
            return carry

        lax.fori_loop(0, n_grp, grp, 0)
        for b in range(NB):
            pltpu.make_async_copy(
                ones_v, deg_sh.at[idx_all.at[0]], sems[b]).wait()
        plsc.subcore_barrier()
        pltpu.sync_copy(deg_sh.at[pl.ds(r0, RPT)],
                        out_hbm.at[c, pl.ds(r0, RPT)])

    return deg_kernel(dst2)


def _sc_propagate(hs_pad, src2, dst2, zeros2):
    total_chunks = src2.shape[0]
    k_ch = total_chunks // NW
    n_grp = k_ch // NB
    mesh = plsc.VectorSubcoreMesh(core_axis_name="c", subcore_axis_name="s")

    @functools.partial(
        pl.kernel,
        out_type=jax.ShapeDtypeStruct((NC, N_PAD, D_HID), jnp.float32),
        mesh=mesh,
        scratch_types=[
            pltpu.VMEM((k_ch, CHUNK), jnp.int32),
            pltpu.VMEM((k_ch, CHUNK), jnp.int32),
            pltpu.VMEM((NB, CHUNK, D_HID), jnp.float32),
            pltpu.VMEM_SHARED((N_PAD, D_HID), jnp.float32),
        ] + [pltpu.SemaphoreType.DMA] * (2 * NB),
        compiler_params=pltpu.CompilerParams(use_tc_tiling_on_sc=False),
    )
    def prop_kernel(hs_hbm, src_hbm, dst_hbm, zeros_hbm, out_hbm,
                    sidx, didx, rows, acc_sh, *sems):
        gsem = sems[:NB]
        ssem = sems[NB:]
        c = lax.axis_index("c")
        s = lax.axis_index("s")
        ch0 = (c * NS + s) * k_ch
        r0 = s * RPT
        pltpu.sync_copy(src_hbm.at[pl.ds(ch0, k_ch)], sidx)
        pltpu.sync_copy(dst_hbm.at[pl.ds(ch0, k_ch)], didx)
        pltpu.sync_copy(zeros_hbm.at[pl.ds(r0, RPT)], acc_sh.at[pl.ds(r0, RPT)])
        plsc.subcore_barrier()

        def start_gather(k, b):
            pltpu.async_copy(hs_hbm.at[sidx.at[k]], rows.at[b], gsem[b])

        def wait_gather(b):
            pltpu.make_async_copy(
                hs_hbm.at[sidx.at[0]], rows.at[b], gsem[b]).wait()

        def start_scatter(k, b):
            pltpu.async_copy(rows.at[b], acc_sh.at[didx.at[k]], ssem[b],
                             add=True)

        def wait_scatter(b):
            pltpu.make_async_copy(
                rows.at[b], acc_sh.at[didx.at[0]], ssem[b]).wait()

        for b in range(NB):
            start_gather(b, b)

        def grp(g, carry):
            for b in range(NB):
                wait_gather(b)
                start_scatter(g * NB + b, b)
            for b in range(NB):
                wait_scatter(b)

                @pl.when(g + 1 < n_grp)
                def _():
                    start_gather((g + 1) * NB + b, b)
            return carry

        lax.fori_loop(0, n_grp, grp, 0)
        plsc.subcore_barrier()
        pltpu.sync_copy(acc_sh.at[pl.ds(r0, RPT)],
                        out_hbm.at[c, pl.ds(r0, RPT)])

    return prop_kernel(hs_pad, src2, dst2, zeros2)


def _tc_first(deg_parts_t, x_pad, W1):
    def body(deg_ref, x_ref, w_ref, hs_ref, dinv_ref):
        deg = deg_ref[...]
        degt = deg[:, 0:1] + deg[:, 1:2] + 1.0
        dinv = lax.rsqrt(jnp.maximum(degt, 1e-12))
        h = jnp.dot(x_ref[...], w_ref[...], preferred_element_type=jnp.float32)
        hs_ref[...] = dinv * h
        dinv_ref[...] = dinv

    return pl.pallas_call(
        body,
        grid=(GRID_M,),
        in_specs=[
            pl.BlockSpec((MB, NC), lambda i: (i, 0)),
            pl.BlockSpec((MB, D_FEAT), lambda i: (i, 0)),
            pl.BlockSpec((D_FEAT, D_HID), lambda i: (0, 0)),
        ],
        out_specs=[
            pl.BlockSpec((MB, D_HID), lambda i: (i, 0)),
            pl.BlockSpec((MB, 1), lambda i: (i, 0)),
        ],
        out_shape=[
            jax.ShapeDtypeStruct((N_PAD, D_HID), jnp.float32),
            jax.ShapeDtypeStruct((N_PAD, 1), jnp.float32),
        ],
    )(deg_parts_t, x_pad, W1)


def _tc_mid(acc_parts, hs1, dinv, b1, W2):
    def body(acc_ref, hs_ref, dinv_ref, b_ref, w_ref, out_ref):
        agg = acc_ref[0] + acc_ref[1] + hs_ref[...]
        pre = dinv_ref[...] * agg + b_ref[...]
        h = jnp.maximum(pre, 0.0)
        h2 = jnp.dot(h, w_ref[...], preferred_element_type=jnp.float32)
        out_ref[...] = dinv_ref[...] * h2

    return pl.pallas_call(
        body,
        grid=(GRID_M,),
        in_specs=[
            pl.BlockSpec((NC, MB, D_HID), lambda i: (0, i, 0)),
            pl.BlockSpec((MB, D_HID), lambda i: (i, 0)),
            pl.BlockSpec((MB, 1), lambda i: (i, 0)),
            pl.BlockSpec((1, D_HID), lambda i: (0, 0)),
            pl.BlockSpec((D_HID, D_HID), lambda i: (0, 0)),
        ],
        out_specs=pl.BlockSpec((MB, D_HID), lambda i: (i, 0)),
        out_shape=jax.ShapeDtypeStruct((N_PAD, D_HID), jnp.float32),
    )(acc_parts, hs1, dinv, b1, W2)


def _tc_final(acc_parts, hs2, dinv, b2):
    def body(acc_ref, hs_ref, dinv_ref, b_ref, out_ref):
        agg = acc_ref[0] + acc_ref[1] + hs_ref[...]
        out_ref[...] = dinv_ref[...] * agg + b_ref[...]

    return pl.pallas_call(
        body,
        grid=(GRID_M,),
        in_specs=[
            pl.BlockSpec((NC, MB, D_HID), lambda i: (0, i, 0)),
            pl.BlockSpec((MB, D_HID), lambda i: (i, 0)),
            pl.BlockSpec((MB, 1), lambda i: (i, 0)),
            pl.BlockSpec((1, D_HID), lambda i: (0, 0)),
        ],
        out_specs=pl.BlockSpec((MB, D_HID), lambda i: (i, 0)),
        out_shape=jax.ShapeDtypeStruct((N_PAD, D_HID), jnp.float32),
    )(acc_parts, hs2, dinv, b2)


def kernel(features, edge_index, W1, b1, W2, b2):
    src = edge_index[0].astype(jnp.int32)
    dst = edge_index[1].astype(jnp.int32)
    e = src.shape[0]
    quantum = NW * CHUNK
    k_ch = -(-e // quantum)
    k_ch = -(-k_ch // NB) * NB
    e_pad = k_ch * quantum
    fill = (N_NODES + jnp.arange(e_pad - e, dtype=jnp.int32)
            % (N_PAD - N_NODES))
    src2 = jnp.concatenate([src, fill]).reshape(NW * k_ch, CHUNK)
    dst2 = jnp.concatenate([dst, fill]).reshape(NW * k_ch, CHUNK)

    x_pad = jnp.pad(features, ((0, N_PAD - N_NODES), (0, 0)))
    zeros2 = jnp.zeros((N_PAD, D_HID), jnp.float32)
    b1r = b1.reshape(1, D_HID)
    b2r = b2.reshape(1, D_HID)

    deg_parts = _sc_degree(dst2)
    hs1, dinv = _tc_first(deg_parts.T, x_pad, W1)
    acc1 = _sc_propagate(hs1, src2, dst2, zeros2)
    hs2 = _tc_mid(acc1, hs1, dinv, b1r, W2)
    acc2 = _sc_propagate(hs2, src2, dst2, zeros2)
    out = _tc_final(acc2, hs2, dinv, b2r)
    return out[:N_NODES]


# speedup vs baseline: 1.4024x; 1.0048x over previous
import functools

import jax
import jax.numpy as jnp
from jax import lax
from jax.experimental import pallas as pl
from jax.experimental.pallas import tpu as pltpu
from jax.experimental.pallas import tpu_sc as plsc

N_NODES = 10000
D_FEAT = 128
D_HID = 64
NC, NS = 2, 16
NW = NC * NS
N_PAD = 10240
RPT = N_PAD // NS
CHUNK = 128
NB = 8

MB = 2560
GRID_M = N_PAD // MB


def _sc_degree(dst2):
    total_chunks = dst2.shape[0]
    k_ch = total_chunks // NW
    n_grp = k_ch // NB
    mesh = plsc.VectorSubcoreMesh(core_axis_name="c", subcore_axis_name="s")

    @functools.partial(
        pl.kernel,
        out_type=jax.ShapeDtypeStruct((NC, N_PAD), jnp.float32),
        mesh=mesh,
        scratch_types=[
            pltpu.VMEM((k_ch, CHUNK), jnp.int32),
            pltpu.VMEM((CHUNK,), jnp.float32),
            pltpu.VMEM_SHARED((N_PAD,), jnp.float32),
            pltpu.VMEM((RPT,), jnp.float32),
        ] + [pltpu.SemaphoreType.DMA] * NB,
    )
    def deg_kernel(dst_hbm, out_hbm, idx_all, ones_v, deg_sh, zero_v, *sems):
        c = lax.axis_index("c")
        s = lax.axis_index("s")
        ch0 = (c * NS + s) * k_ch
        r0 = s * RPT
        for i in range(CHUNK // 16):
            ones_v[pl.ds(i * 16, 16)] = jnp.ones((16,), jnp.float32)
        for i in range(RPT // 16):
            zero_v[pl.ds(i * 16, 16)] = jnp.zeros((16,), jnp.float32)
        pltpu.sync_copy(dst_hbm.at[pl.ds(ch0, k_ch)], idx_all)
        pltpu.sync_copy(zero_v, deg_sh.at[pl.ds(r0, RPT)])
        plsc.subcore_barrier()

        def scat(k, b):
            pltpu.async_copy(ones_v, deg_sh.at[idx_all.at[k]], sems[b],
                             add=True)

        def grp(g, carry):
            for b in range(NB):
                @pl.when(g > 0)
                def _():
                    pltpu.make_async_copy(
                        ones_v, deg_sh.at[idx_all.at[0]], sems[b]).wait()
                scat(g * NB + b, b)
            return carry

        lax.fori_loop(0, n_grp, grp, 0)
        for b in range(NB):
            pltpu.make_async_copy(
                ones_v, deg_sh.at[idx_all.at[0]], sems[b]).wait()
        plsc.subcore_barrier()
        pltpu.sync_copy(deg_sh.at[pl.ds(r0, RPT)],
                        out_hbm.at[c, pl.ds(r0, RPT)])

    return deg_kernel(dst2)


def _sc_propagate(hs_pad, src2, dst2, zeros2):
    total_chunks = src2.shape[0]
    k_ch = total_chunks // NW
    n_grp = k_ch // NB
    mesh = plsc.VectorSubcoreMesh(core_axis_name="c", subcore_axis_name="s")

    @functools.partial(
        pl.kernel,
        out_type=jax.ShapeDtypeStruct((NC, N_PAD, D_HID), jnp.float32),
        mesh=mesh,
        scratch_types=[
            pltpu.VMEM((k_ch, CHUNK), jnp.int32),
            pltpu.VMEM((k_ch, CHUNK), jnp.int32),
            pltpu.VMEM((NB, CHUNK, D_HID), jnp.float32),
            pltpu.VMEM_SHARED((N_PAD, D_HID), jnp.float32),
        ] + [pltpu.SemaphoreType.DMA] * (2 * NB),
        compiler_params=pltpu.CompilerParams(use_tc_tiling_on_sc=False),
    )
    def prop_kernel(hs_hbm, src_hbm, dst_hbm, zeros_hbm, out_hbm,
                    sidx, didx, rows, acc_sh, *sems):
        gsem = sems[:NB]
        ssem = sems[NB:]
        c = lax.axis_index("c")
        s = lax.axis_index("s")
        ch0 = (c * NS + s) * k_ch
        r0 = s * RPT
        pltpu.sync_copy(src_hbm.at[pl.ds(ch0, k_ch)], sidx)
        pltpu.sync_copy(dst_hbm.at[pl.ds(ch0, k_ch)], didx)
        pltpu.sync_copy(zeros_hbm.at[pl.ds(r0, RPT)], acc_sh.at[pl.ds(r0, RPT)])
        plsc.subcore_barrier()

        def start_gather(k, b):
            pltpu.async_copy(hs_hbm.at[sidx.at[k]], rows.at[b], gsem[b])

        def wait_gather(b):
            pltpu.make_async_copy(
                hs_hbm.at[sidx.at[0]], rows.at[b], gsem[b]).wait()

        def start_scatter(k, b):
            pltpu.async_copy(rows.at[b], acc_sh.at[didx.at[k]], ssem[b],
                             add=True)

        def wait_scatter(b):
            pltpu.make_async_copy(
                rows.at[b], acc_sh.at[didx.at[0]], ssem[b]).wait()

        for b in range(NB):
            start_gather(b, b)

        def grp(g, carry):
            for b in range(NB):
                wait_gather(b)
                start_scatter(g * NB + b, b)
            for b in range(NB):
                wait_scatter(b)

                @pl.when(g + 1 < n_grp)
                def _():
                    start_gather((g + 1) * NB + b, b)
            return carry

        lax.fori_loop(0, n_grp, grp, 0)
        plsc.subcore_barrier()
        pltpu.sync_copy(acc_sh.at[pl.ds(r0, RPT)],
                        out_hbm.at[c, pl.ds(r0, RPT)])

    return prop_kernel(hs_pad, src2, dst2, zeros2)


def _tc_first(deg_parts_t, x_pad, W1):
    def body(deg_ref, x_ref, w_ref, hs_ref, dinv_ref):
        deg = deg_ref[...]
        degt = deg[:, 0:1] + deg[:, 1:2] + 1.0
        dinv = lax.rsqrt(jnp.maximum(degt, 1e-12))
        h = jnp.dot(x_ref[...], w_ref[...], preferred_element_type=jnp.float32)
        hs_ref[...] = dinv * h
        dinv_ref[...] = dinv

    return pl.pallas_call(
        body,
        grid=(GRID_M,),
        in_specs=[
            pl.BlockSpec((MB, NC), lambda i: (i, 0)),
            pl.BlockSpec((MB, D_FEAT), lambda i: (i, 0)),
            pl.BlockSpec((D_FEAT, D_HID), lambda i: (0, 0)),
        ],
        out_specs=[
            pl.BlockSpec((MB, D_HID), lambda i: (i, 0)),
            pl.BlockSpec((MB, 1), lambda i: (i, 0)),
        ],
        out_shape=[
            jax.ShapeDtypeStruct((N_PAD, D_HID), jnp.float32),
            jax.ShapeDtypeStruct((N_PAD, 1), jnp.float32),
        ],
    )(deg_parts_t, x_pad, W1)


def _tc_mid(acc_parts, hs1, dinv, b1, W2):
    def body(acc_ref, hs_ref, dinv_ref, b_ref, w_ref, out_ref):
        agg = acc_ref[0] + acc_ref[1] + hs_ref[...]
        pre = dinv_ref[...] * agg + b_ref[...]
        h = jnp.maximum(pre, 0.0)
        h2 = jnp.dot(h, w_ref[...], preferred_element_type=jnp.float32)
        out_ref[...] = dinv_ref[...] * h2

    return pl.pallas_call(
        body,
        grid=(GRID_M,),
        in_specs=[
            pl.BlockSpec((NC, MB, D_HID), lambda i: (0, i, 0)),
            pl.BlockSpec((MB, D_HID), lambda i: (i, 0)),
            pl.BlockSpec((MB, 1), lambda i: (i, 0)),
            pl.BlockSpec((1, D_HID), lambda i: (0, 0)),
            pl.BlockSpec((D_HID, D_HID), lambda i: (0, 0)),
        ],
        out_specs=pl.BlockSpec((MB, D_HID), lambda i: (i, 0)),
        out_shape=jax.ShapeDtypeStruct((N_PAD, D_HID), jnp.float32),
    )(acc_parts, hs1, dinv, b1, W2)


def _tc_final(acc_parts, hs2, dinv, b2):
    def body(acc_ref, hs_ref, dinv_ref, b_ref, out_ref):
        agg = acc_ref[0] + acc_ref[1] + hs_ref[...]
        out_ref[...] = dinv_ref[...] * agg + b_ref[...]

    mf = 2000
    return pl.pallas_call(
        body,
        grid=(N_NODES // mf,),
        in_specs=[
            pl.BlockSpec((NC, mf, D_HID), lambda i: (0, i, 0)),
            pl.BlockSpec((mf, D_HID), lambda i: (i, 0)),
            pl.BlockSpec((mf, 1), lambda i: (i, 0)),
            pl.BlockSpec((1, D_HID), lambda i: (0, 0)),
        ],
        out_specs=pl.BlockSpec((mf, D_HID), lambda i: (i, 0)),
        out_shape=jax.ShapeDtypeStruct((N_NODES, D_HID), jnp.float32),
    )(acc_parts, hs2, dinv, b2)


def kernel(features, edge_index, W1, b1, W2, b2):
    src = edge_index[0].astype(jnp.int32)
    dst = edge_index[1].astype(jnp.int32)
    e = src.shape[0]
    quantum = NW * CHUNK
    k_ch = -(-e // quantum)
    k_ch = -(-k_ch // NB) * NB
    e_pad = k_ch * quantum
    fill = (N_NODES + jnp.arange(e_pad - e, dtype=jnp.int32)
            % (N_PAD - N_NODES))
    src2 = jnp.concatenate([src, fill]).reshape(NW * k_ch, CHUNK)
    dst2 = jnp.concatenate([dst, fill]).reshape(NW * k_ch, CHUNK)

    x_pad = jnp.pad(features, ((0, N_PAD - N_NODES), (0, 0)))
    zeros2 = jnp.zeros((N_PAD, D_HID), jnp.float32)
    b1r = b1.reshape(1, D_HID)
    b2r = b2.reshape(1, D_HID)

    deg_parts = _sc_degree(dst2)
    hs1, dinv = _tc_first(deg_parts.T, x_pad, W1)
    acc1 = _sc_propagate(hs1, src2, dst2, zeros2)
    hs2 = _tc_mid(acc1, hs1, dinv, b1r, W2)
    acc2 = _sc_propagate(hs2, src2, dst2, zeros2)
    return _tc_final(acc2, hs2, dinv, b2r)
